# Initial kernel scaffold; baseline (speedup 1.0000x reference)
import jax
import jax.numpy as jnp
from jax.experimental import pallas as pl


def kernel(x, edge_index, batch, params):
    raise NotImplementedError("write your pallas kernel here")


# speedup vs baseline: 1.4287x; 1.4287x over previous
import functools

import jax
import jax.numpy as jnp
from jax import lax
from jax.experimental import pallas as pl
from jax.experimental.pallas import tpu as pltpu
from jax.experimental.pallas import tpu_sc as plsc

_N = 10000
_E = 320000
_ALPHA = 0.95
_EPS = 1e-5

_NC = 2
_NS = 16
_NW = _NC * _NS
_EB = 128
_NB = 80
_EPW = _EB * _NB
_EPAD = _EPW * _NW
_NSP = 10240
_RPS = _NSP // _NS
_ZR = 32


@functools.lru_cache(maxsize=None)
def _make_seg_sum(n_chunks):
    mesh = plsc.VectorSubcoreMesh(core_axis_name="c", subcore_axis_name="s")
    out_type = jax.ShapeDtypeStruct((_NC, n_chunks, _NSP, 128), jnp.float32)
    scratch = [
        pltpu.VMEM((_NB, _EB), jnp.int32),
        pltpu.VMEM((_NB, _EB), jnp.int32),
        pltpu.VMEM((_EB, 128), jnp.float32),
        pltpu.VMEM((_ZR, 128), jnp.float32),
        pltpu.VMEM_SHARED((_NSP, 128), jnp.float32),
        pltpu.SemaphoreType.DMA,
    ]

    def body(x_cm, sidx_h, didx_h, zeros_h, part_h,
             sidx_v, didx_v, rows_v, zb_v, agg_sp, sem):
        cid = lax.axis_index("c")
        sid = lax.axis_index("s")
        wid = cid * _NS + sid
        pltpu.sync_copy(didx_h.at[wid], didx_v)
        pltpu.sync_copy(zeros_h, zb_v)

        for c in range(n_chunks):
            def zbody(z, carry):
                pltpu.sync_copy(zb_v, agg_sp.at[pl.ds(sid * _RPS + z * _ZR, _ZR)])
                return carry

            lax.fori_loop(0, _RPS // _ZR, zbody, 0)
            plsc.subcore_barrier()
            pltpu.sync_copy(sidx_h.at[c, wid], sidx_v)

            def ebody(j, carry):
                pltpu.async_copy(x_cm.at[sidx_v.at[j]], rows_v, sem).wait()
                pltpu.sync_copy(rows_v, agg_sp.at[didx_v.at[j]], add=True)
                return carry

            lax.fori_loop(0, _NB, ebody, 0)
            plsc.subcore_barrier()
            pltpu.sync_copy(agg_sp.at[pl.ds(sid * _RPS, _RPS)],
                            part_h.at[cid, c, pl.ds(sid * _RPS, _RPS)])

    return pl.kernel(body, out_type=out_type, mesh=mesh,
                     scratch_types=scratch)


@functools.lru_cache(maxsize=None)
def _make_deg():
    mesh = plsc.VectorSubcoreMesh(core_axis_name="c", subcore_axis_name="s")
    out_type = jax.ShapeDtypeStruct((_NC, _NSP, 16), jnp.float32)
    scratch = [
        pltpu.VMEM((_NB, _EB), jnp.int32),
        pltpu.VMEM((_EB, 16), jnp.float32),
        pltpu.VMEM((64, 16), jnp.float32),
        pltpu.VMEM_SHARED((_NSP, 16), jnp.float32),
    ]

    def body(didx_h, ones_h, zeros_h, deg_h, didx_v, ones_v, zd_v, deg_sp):
        cid = lax.axis_index("c")
        sid = lax.axis_index("s")
        wid = cid * _NS + sid
        pltpu.sync_copy(didx_h.at[wid], didx_v)
        pltpu.sync_copy(ones_h, ones_v)
        pltpu.sync_copy(zeros_h, zd_v)

        def zbody(z, carry):
            pltpu.sync_copy(zd_v, deg_sp.at[pl.ds(sid * _RPS + z * 64, 64)])
            return carry

        lax.fori_loop(0, _RPS // 64, zbody, 0)
        plsc.subcore_barrier()

        def dbody(j, carry):
            pltpu.sync_copy(ones_v, deg_sp.at[didx_v.at[j]], add=True)
            return carry

        lax.fori_loop(0, _NB, dbody, 0)
        plsc.subcore_barrier()
        pltpu.sync_copy(deg_sp.at[pl.ds(sid * _RPS, _RPS)],
                        deg_h.at[cid, pl.ds(sid * _RPS, _RPS)])

    return pl.kernel(body, out_type=out_type, mesh=mesh,
                     scratch_types=scratch)


def _matmul(x, w, b, *, x2=None, w2=None, scale=None, shift=None,
            act=None, stats=False):
    M, K = x.shape
    Nout = w.shape[1]
    bm = 1000
    bn = min(1024, Nout)
    bk = min(512, K)
    nm, nn, nk = M // bm, Nout // bn, K // bk
    two = x2 is not None
    aff = scale is not None

    in_specs = [pl.BlockSpec((bm, bk), lambda n, m, k: (m, k))]
    inputs = [x]
    if two:
        in_specs.append(pl.BlockSpec((bm, bk), lambda n, m, k: (m, k)))
        inputs.append(x2)
    in_specs.append(pl.BlockSpec((bk, bn), lambda n, m, k: (k, n)))
    inputs.append(w)
    if two:
        in_specs.append(pl.BlockSpec((bk, bn), lambda n, m, k: (k, n)))
        inputs.append(w2)
    in_specs.append(pl.BlockSpec((1, bn), lambda n, m, k: (0, n)))
    inputs.append(b.reshape(1, Nout))
    if aff:
        in_specs.append(pl.BlockSpec((1, bk), lambda n, m, k: (0, k)))
        inputs.append(scale.reshape(1, K))
        in_specs.append(pl.BlockSpec((1, bk), lambda n, m, k: (0, k)))
        inputs.append(shift.reshape(1, K))

    out_shape = [jax.ShapeDtypeStruct((M, Nout), jnp.float32)]
    out_specs = [pl.BlockSpec((bm, bn), lambda n, m, k: (m, n))]
    if stats:
        out_shape += [jax.ShapeDtypeStruct((nm, 1, Nout), jnp.float32)] * 2
        out_specs += [pl.BlockSpec((1, 1, bn), lambda n, m, k: (m, 0, n))] * 2

    def body(*refs):
        it = iter(refs)
        x_ref = next(it)
        x2_ref = next(it) if two else None
        w_ref = next(it)
        w2_ref = next(it) if two else None
        b_ref = next(it)
        sc_ref = next(it) if aff else None
        sh_ref = next(it) if aff else None
        o_ref = next(it)
        s_ref = next(it) if stats else None
        q_ref = next(it) if stats else None
        acc = next(it)

        k = pl.program_id(2)
        xb = x_ref[...]
        if aff:
            xb = jnp.maximum(xb * sc_ref[...] + sh_ref[...], 0.0)
        part = jnp.dot(xb, w_ref[...], preferred_element_type=jnp.float32)
        if two:
            part = part + jnp.dot(x2_ref[...], w2_ref[...],
                                  preferred_element_type=jnp.float32)

        @pl.when(k == 0)
        def _():
            acc[...] = part

        @pl.when(k > 0)
        def _():
            acc[...] += part

        @pl.when(k == nk - 1)
        def _():
            h = acc[...] + b_ref[...]
            if act == "relu":
                o_ref[...] = jnp.maximum(h, 0.0)
            elif act == "sigmoid":
                o_ref[...] = jax.nn.sigmoid(h)
            else:
                o_ref[...] = h
            if stats:
                s_ref[...] = jnp.sum(h, axis=0)[None, None, :]
                q_ref[...] = jnp.sum(h * h, axis=0)[None, None, :]

    return pl.pallas_call(
        body,
        grid=(nn, nm, nk),
        in_specs=in_specs,
        out_specs=out_specs,
        out_shape=out_shape,
        scratch_shapes=[pltpu.VMEM((bm, bn), jnp.float32)],
        compiler_params=pltpu.CompilerParams(
            dimension_semantics=("parallel", "parallel", "arbitrary")),
    )(*inputs)


def kernel(x, edge_index, batch, params):
    p = params
    src = edge_index[0]
    dst = edge_index[1]
    pad = _EPAD - _E
    src_p = jnp.concatenate(
        [src, jnp.zeros((pad,), jnp.int32)]).reshape(_NW, _NB, _EB)
    dst_p = jnp.concatenate(
        [dst, jnp.full((pad,), _N, jnp.int32)]).reshape(_NW, _NB, _EB)
    zeros128 = jnp.zeros((_ZR, 128), jnp.float32)
    ones16 = jnp.ones((_EB, 16), jnp.float32)
    zeros16 = jnp.zeros((64, 16), jnp.float32)

    def seg(x_in, n_chunks):
        d = x_in.shape[1]
        x_cm = x_in.reshape(_N, n_chunks, 128).transpose(1, 0, 2)
        x_cm = x_cm.reshape(n_chunks * _N, 128)
        sidx = (src_p[None]
                + (jnp.arange(n_chunks, dtype=jnp.int32) * _N)[:, None, None, None])
        part = _make_seg_sum(n_chunks)(x_cm, sidx, dst_p, zeros128)
        agg = (part[0] + part[1])[:, :_N, :]
        return agg.transpose(1, 0, 2).reshape(_N, d)

    def block(xb, agg, invd, i):
        mean = agg * invd[:, None]
        ws = _ALPHA * p['W%ds' % i]
        wn = (1.0 - _ALPHA) * p['W%dn' % i]
        h, s, q = _matmul(xb, ws, p['b%d' % i], x2=mean, w2=wn, stats=True)
        mu = jnp.sum(s, axis=(0, 1)) / _N
        var = jnp.sum(q, axis=(0, 1)) / _N - mu * mu
        bn_scale = p['g%d' % i] * lax.rsqrt(var + _EPS)
        bn_shift = p['be%d' % i] - mu * bn_scale
        y, = _matmul(h, p['L%dw' % i], p['L%db' % i],
                     scale=bn_scale, shift=bn_shift, act="relu")
        return y

    degp = _make_deg()(dst_p, ones16, zeros16)
    deg = degp[0, :_N, 0] + degp[1, :_N, 0]
    agg1 = seg(x, 1)
    invd = 1.0 / jnp.clip(deg, 1.0, None)
    y = block(x, agg1, invd, 1)
    y = block(y, seg(y, 4), invd, 2)
    y = block(y, seg(y, 8), invd, 3)
    z, = _matmul(y, p['F1w'], p['F1b'])
    z, = _matmul(z, p['F2w'], p['F2b'])
    out, = _matmul(z, p['Ow'], p['Ob'], act="sigmoid")
    return out


# speedup vs baseline: 1.6765x; 1.1734x over previous
import functools

import jax
import jax.numpy as jnp
from jax import lax
from jax.experimental import pallas as pl
from jax.experimental.pallas import tpu as pltpu
from jax.experimental.pallas import tpu_sc as plsc

_N = 10000
_E = 320000
_ALPHA = 0.95
_EPS = 1e-5

_NC = 2
_NS = 16
_NW = _NC * _NS
_EB = 128
_NB = 80
_EPW = _EB * _NB
_EPAD = _EPW * _NW
_NSP = 10240
_RPS = _NSP // _NS
_ZR = 32

_HB = _NB // 2


@functools.lru_cache(maxsize=None)
def _make_seg_sum(n_chunks):
    mesh = plsc.VectorSubcoreMesh(core_axis_name="c", subcore_axis_name="s")
    out_type = jax.ShapeDtypeStruct((_NC, n_chunks, _NSP, 128), jnp.float32)
    scratch = [
        pltpu.VMEM((_HB, _EB), jnp.int32),
        pltpu.VMEM((_HB, _EB), jnp.int32),
        pltpu.VMEM((_EB, 128), jnp.float32),
        pltpu.VMEM((_EB, 128), jnp.float32),
        pltpu.VMEM((_ZR, 128), jnp.float32),
        pltpu.VMEM_SHARED((_NSP, 128), jnp.float32),
        pltpu.SemaphoreType.DMA,
        pltpu.SemaphoreType.DMA,
    ]

    def body(x_cm, sidx_h, didx_h, zeros_h, part_h,
             sidx_v, didx_v, rows_v0, rows_v1, zb_v, agg_sp, sem0, sem1):
        cid = lax.axis_index("c")
        sid = lax.axis_index("s")
        wid = cid * _NS + sid
        pltpu.sync_copy(zeros_h, zb_v)
        bufs = ((rows_v0, sem0), (rows_v1, sem1))

        for c in range(n_chunks):
            def zbody(z, carry):
                pltpu.sync_copy(zb_v, agg_sp.at[pl.ds(sid * _RPS + z * _ZR, _ZR)])
                return carry

            lax.fori_loop(0, _RPS // _ZR, zbody, 0)
            plsc.subcore_barrier()

            for h in range(2):
                pltpu.sync_copy(sidx_h.at[2 * c + h, wid], sidx_v)
                pltpu.sync_copy(didx_h.at[h, wid], didx_v)
                pltpu.async_copy(x_cm.at[sidx_v.at[0]], rows_v0, sem0)
                pltpu.async_copy(x_cm.at[sidx_v.at[1]], rows_v1, sem1)

                def gbody(g, carry):
                    for b in range(2):
                        j = g * 2 + b
                        rv, sem = bufs[b]
                        pltpu.make_async_copy(
                            x_cm.at[sidx_v.at[j]], rv, sem).wait()
                        pltpu.sync_copy(rv, agg_sp.at[didx_v.at[j]], add=True)
                        pltpu.async_copy(x_cm.at[sidx_v.at[j + 2]], rv, sem)
                    return carry

                lax.fori_loop(0, _HB // 2 - 1, gbody, 0)
                for b in range(2):
                    j = _HB - 2 + b
                    rv, sem = bufs[b]
                    pltpu.make_async_copy(
                        x_cm.at[sidx_v.at[j]], rv, sem).wait()
                    pltpu.sync_copy(rv, agg_sp.at[didx_v.at[j]], add=True)

            plsc.subcore_barrier()
            pltpu.sync_copy(agg_sp.at[pl.ds(sid * _RPS, _RPS)],
                            part_h.at[cid, c, pl.ds(sid * _RPS, _RPS)])

    return pl.kernel(body, out_type=out_type, mesh=mesh,
                     scratch_types=scratch)


@functools.lru_cache(maxsize=None)
def _make_deg():
    mesh = plsc.VectorSubcoreMesh(core_axis_name="c", subcore_axis_name="s")
    out_type = jax.ShapeDtypeStruct((_NC, _NSP, 128), jnp.float32)
    scratch = [
        pltpu.VMEM((_NB, _EB), jnp.int32),
        pltpu.VMEM((_EB, 128), jnp.float32),
        pltpu.VMEM((_ZR, 128), jnp.float32),
        pltpu.VMEM_SHARED((_NSP, 128), jnp.float32),
    ]

    def body(didx_h, ones_h, zeros_h, deg_h, didx_v, ones_v, zd_v, deg_sp):
        cid = lax.axis_index("c")
        sid = lax.axis_index("s")
        wid = cid * _NS + sid
        pltpu.sync_copy(didx_h.at[wid], didx_v)
        pltpu.sync_copy(ones_h, ones_v)
        pltpu.sync_copy(zeros_h, zd_v)

        def zbody(z, carry):
            pltpu.sync_copy(zd_v, deg_sp.at[pl.ds(sid * _RPS + z * _ZR, _ZR)])
            return carry

        lax.fori_loop(0, _RPS // _ZR, zbody, 0)
        plsc.subcore_barrier()

        def dbody(j, carry):
            pltpu.sync_copy(ones_v, deg_sp.at[didx_v.at[j]], add=True)
            return carry

        lax.fori_loop(0, _NB, dbody, 0)
        plsc.subcore_barrier()
        pltpu.sync_copy(deg_sp.at[pl.ds(sid * _RPS, _RPS)],
                        deg_h.at[cid, pl.ds(sid * _RPS, _RPS)])

    return pl.kernel(body, out_type=out_type, mesh=mesh,
                     scratch_types=scratch)


def _matmul(x, w, b, *, x2=None, w2=None, scale=None, shift=None,
            act=None, stats=False):
    M, K = x.shape
    Nout = w.shape[1]
    bm = 1000
    bn = min(1024, Nout)
    bk = min(512, K)
    nm, nn, nk = M // bm, Nout // bn, K // bk
    two = x2 is not None
    aff = scale is not None

    in_specs = [pl.BlockSpec((bm, bk), lambda n, m, k: (m, k))]
    inputs = [x]
    if two:
        in_specs.append(pl.BlockSpec((bm, bk), lambda n, m, k: (m, k)))
        inputs.append(x2)
    in_specs.append(pl.BlockSpec((bk, bn), lambda n, m, k: (k, n)))
    inputs.append(w)
    if two:
        in_specs.append(pl.BlockSpec((bk, bn), lambda n, m, k: (k, n)))
        inputs.append(w2)
    in_specs.append(pl.BlockSpec((1, bn), lambda n, m, k: (0, n)))
    inputs.append(b.reshape(1, Nout))
    if aff:
        in_specs.append(pl.BlockSpec((1, bk), lambda n, m, k: (0, k)))
        inputs.append(scale.reshape(1, K))
        in_specs.append(pl.BlockSpec((1, bk), lambda n, m, k: (0, k)))
        inputs.append(shift.reshape(1, K))

    out_shape = [jax.ShapeDtypeStruct((M, Nout), jnp.float32)]
    out_specs = [pl.BlockSpec((bm, bn), lambda n, m, k: (m, n))]
    if stats:
        out_shape += [jax.ShapeDtypeStruct((nm, 1, Nout), jnp.float32)] * 2
        out_specs += [pl.BlockSpec((1, 1, bn), lambda n, m, k: (m, 0, n))] * 2

    def body(*refs):
        it = iter(refs)
        x_ref = next(it)
        x2_ref = next(it) if two else None
        w_ref = next(it)
        w2_ref = next(it) if two else None
        b_ref = next(it)
        sc_ref = next(it) if aff else None
        sh_ref = next(it) if aff else None
        o_ref = next(it)
        s_ref = next(it) if stats else None
        q_ref = next(it) if stats else None
        acc = next(it)

        k = pl.program_id(2)
        xb = x_ref[...]
        if aff:
            xb = jnp.maximum(xb * sc_ref[...] + sh_ref[...], 0.0)
        part = jnp.dot(xb, w_ref[...], preferred_element_type=jnp.float32)
        if two:
            part = part + jnp.dot(x2_ref[...], w2_ref[...],
                                  preferred_element_type=jnp.float32)

        @pl.when(k == 0)
        def _():
            acc[...] = part

        @pl.when(k > 0)
        def _():
            acc[...] += part

        @pl.when(k == nk - 1)
        def _():
            h = acc[...] + b_ref[...]
            if act == "relu":
                o_ref[...] = jnp.maximum(h, 0.0)
            elif act == "sigmoid":
                o_ref[...] = jax.nn.sigmoid(h)
            else:
                o_ref[...] = h
            if stats:
                s_ref[...] = jnp.sum(h, axis=0)[None, None, :]
                q_ref[...] = jnp.sum(h * h, axis=0)[None, None, :]

    return pl.pallas_call(
        body,
        grid=(nn, nm, nk),
        in_specs=in_specs,
        out_specs=out_specs,
        out_shape=out_shape,
        scratch_shapes=[pltpu.VMEM((bm, bn), jnp.float32)],
        compiler_params=pltpu.CompilerParams(
            dimension_semantics=("parallel", "parallel", "arbitrary")),
    )(*inputs)


def kernel(x, edge_index, batch, params):
    p = params
    src = edge_index[0]
    dst = edge_index[1]
    pad = _EPAD - _E
    src_p = jnp.swapaxes(jnp.concatenate(
        [src, jnp.zeros((pad,), jnp.int32)]).reshape(_NW, 2, _HB, _EB), 0, 1)
    dst_flat = jnp.concatenate(
        [dst, jnp.full((pad,), _N, jnp.int32)]).reshape(_NW, _NB, _EB)
    dst_p = jnp.swapaxes(dst_flat.reshape(_NW, 2, _HB, _EB), 0, 1)
    zeros128 = jnp.zeros((_ZR, 128), jnp.float32)
    ones128 = jnp.ones((_EB, 128), jnp.float32)

    def seg(x_in, n_chunks):
        d = x_in.shape[1]
        x_cm = x_in.reshape(_N, n_chunks, 128).transpose(1, 0, 2)
        x_cm = x_cm.reshape(n_chunks * _N, 128)
        sidx = (src_p[None]
                + (jnp.arange(n_chunks, dtype=jnp.int32)
                   * _N)[:, None, None, None, None])
        sidx = sidx.reshape(n_chunks * 2, _NW, _HB, _EB)
        part = _make_seg_sum(n_chunks)(x_cm, sidx, dst_p, zeros128)
        agg = (part[0] + part[1])[:, :_N, :]
        return agg.transpose(1, 0, 2).reshape(_N, d)

    def block(xb, agg, invd, i):
        mean = agg * invd[:, None]
        ws = _ALPHA * p['W%ds' % i]
        wn = (1.0 - _ALPHA) * p['W%dn' % i]
        h, s, q = _matmul(xb, ws, p['b%d' % i], x2=mean, w2=wn, stats=True)
        mu = jnp.sum(s, axis=(0, 1)) / _N
        var = jnp.sum(q, axis=(0, 1)) / _N - mu * mu
        bn_scale = p['g%d' % i] * lax.rsqrt(var + _EPS)
        bn_shift = p['be%d' % i] - mu * bn_scale
        y, = _matmul(h, p['L%dw' % i], p['L%db' % i],
                     scale=bn_scale, shift=bn_shift, act="relu")
        return y

    degp = _make_deg()(dst_flat, ones128, zeros128)
    deg = degp[0, :_N, 0] + degp[1, :_N, 0]
    agg1 = seg(x, 1)
    invd = 1.0 / jnp.clip(deg, 1.0, None)
    y = block(x, agg1, invd, 1)
    y = block(y, seg(y, 4), invd, 2)
    y = block(y, seg(y, 8), invd, 3)
    z, = _matmul(y, p['F1w'], p['F1b'])
    z, = _matmul(z, p['F2w'], p['F2b'])
    out, = _matmul(z, p['Ow'], p['Ob'], act="sigmoid")
    return out
